# baseline (speedup 1.0000x reference)
import jax
import jax.numpy as jnp
from jax.experimental import pallas as pl
from jax.experimental.pallas import tpu as pltpu


_NC = 3
_OFFS = (0, 3336, 6672)
_SIZES = (3336, 3336, 3328)


def _copy_dma(x_hbm, o_hbm, buf, in_sems, out_sems):
    for c in range(_NC):
        pltpu.make_async_copy(
            x_hbm.at[pl.ds(_OFFS[c], _SIZES[c])],
            buf.at[c, pl.ds(0, _SIZES[c])],
            in_sems.at[c],
        ).start()
    for c in range(_NC):
        pltpu.make_async_copy(
            x_hbm.at[pl.ds(_OFFS[c], _SIZES[c])],
            buf.at[c, pl.ds(0, _SIZES[c])],
            in_sems.at[c],
        ).wait()
        pltpu.make_async_copy(
            buf.at[c, pl.ds(0, _SIZES[c])],
            o_hbm.at[pl.ds(_OFFS[c], _SIZES[c])],
            out_sems.at[c],
        ).start()
    for c in range(_NC):
        pltpu.make_async_copy(
            buf.at[c, pl.ds(0, _SIZES[c])],
            o_hbm.at[pl.ds(_OFFS[c], _SIZES[c])],
            out_sems.at[c],
        ).wait()


def kernel(atom_features, degree_slice, membership, n_samples, deg_adj_0):
    del degree_slice, membership, n_samples, deg_adj_0
    rows, cols = atom_features.shape
    return pl.pallas_call(
        _copy_dma,
        in_specs=[pl.BlockSpec(memory_space=pltpu.MemorySpace.HBM)],
        out_specs=pl.BlockSpec(memory_space=pltpu.MemorySpace.HBM),
        out_shape=jax.ShapeDtypeStruct(atom_features.shape, atom_features.dtype),
        scratch_shapes=[
            pltpu.VMEM((_NC, 3336, 128), jnp.float32),
            pltpu.SemaphoreType.DMA((_NC,)),
            pltpu.SemaphoreType.DMA((_NC,)),
        ],
    )(atom_features)


# speedup vs baseline: 1.0062x; 1.0062x over previous
import jax
import jax.numpy as jnp
from jax.experimental import pallas as pl
from jax.experimental.pallas import tpu as pltpu


_NC = 4
_CH = 2500


def _copy_dma(x_hbm, o_hbm, buf, in_sems, out_sems):
    for c in range(_NC):
        pltpu.make_async_copy(
            x_hbm.at[pl.ds(c * _CH, _CH)], buf.at[c], in_sems.at[c]
        ).start()
    for c in range(_NC):
        pltpu.make_async_copy(
            x_hbm.at[pl.ds(c * _CH, _CH)], buf.at[c], in_sems.at[c]
        ).wait()
        pltpu.make_async_copy(
            buf.at[c], o_hbm.at[pl.ds(c * _CH, _CH)], out_sems.at[c]
        ).start()
    for c in range(_NC):
        pltpu.make_async_copy(
            buf.at[c], o_hbm.at[pl.ds(c * _CH, _CH)], out_sems.at[c]
        ).wait()


def kernel(atom_features, degree_slice, membership, n_samples, deg_adj_0):
    del degree_slice, membership, n_samples, deg_adj_0
    rows, cols = atom_features.shape
    return pl.pallas_call(
        _copy_dma,
        in_specs=[pl.BlockSpec(memory_space=pltpu.MemorySpace.HBM)],
        out_specs=pl.BlockSpec(memory_space=pltpu.MemorySpace.HBM),
        out_shape=jax.ShapeDtypeStruct(atom_features.shape, atom_features.dtype),
        scratch_shapes=[
            pltpu.VMEM((_NC, _CH, 128), jnp.float32),
            pltpu.SemaphoreType.DMA((_NC,)),
            pltpu.SemaphoreType.DMA((_NC,)),
        ],
    )(atom_features)
